# Initial kernel scaffold; baseline (speedup 1.0000x reference)
import jax
import jax.numpy as jnp
from jax.experimental import pallas as pl


def kernel(inputs, targets):
    raise NotImplementedError("write your pallas kernel here")


# speedup vs baseline: 11.4403x; 11.4403x over previous
import jax
import jax.numpy as jnp
from jax.experimental import pallas as pl
from jax.experimental.pallas import tpu as pltpu

_C = 19
_KEEP = 512
_RB = 32


def _stage1_body(inp_ref, tgt_ref, dist_ref, nll_ref):
    t = tgt_ref[0]
    x = inp_ref[0, 0]
    s_exp = jnp.exp(x)
    s_abs = jnp.abs(x)
    xt = jnp.where(t == 0, x, 0.0)
    for c in range(1, _C):
        x = inp_ref[0, c]
        s_exp = s_exp + jnp.exp(x)
        s_abs = s_abs + jnp.abs(x)
        xt = jnp.where(t == c, x, xt)
    dist_ref[0] = s_abs - jnp.abs(xt) + jnp.abs(xt - 1.0)
    nll_ref[0] = jnp.log(s_exp) - xt


def _stage2_body(dist_ref, nll_ref, out_ref):
    B, H, W = dist_ref.shape
    CH = 32
    nchunk = H // CH

    def count_gt(mid):
        def inner(j, cnt):
            bits = pltpu.bitcast(dist_ref[:, pl.ds(j * CH, CH), :], jnp.int32)
            gt = (bits > mid[:, :, None]).astype(jnp.int32)
            return cnt + jnp.sum(gt, axis=(1, 2), keepdims=False)[:, None]

        return jax.lax.fori_loop(0, nchunk, inner, jnp.zeros((B, 1), jnp.int32))

    def bisect(i, carry):
        lo, hi = carry
        mid = lo + (hi - lo) // 2
        below = count_gt(mid) < _KEEP
        hi = jnp.where(below, mid, hi)
        lo = jnp.where(below, lo, mid)
        return lo, hi

    lo0 = jnp.full((B, 1), -1, jnp.int32)
    hi0 = jnp.full((B, 1), 0x7F800000, jnp.int32)
    _, thr = jax.lax.fori_loop(0, 31, bisect, (lo0, hi0))

    def stats(j, carry):
        m, s_gt, c_eq, s_eq = carry
        bits = pltpu.bitcast(dist_ref[:, pl.ds(j * CH, CH), :], jnp.int32)
        nll = nll_ref[:, pl.ds(j * CH, CH), :]
        gt = bits > thr[:, :, None]
        eq = bits == thr[:, :, None]
        m = m + jnp.sum(gt.astype(jnp.float32), axis=(1, 2))[:, None]
        s_gt = s_gt + jnp.sum(jnp.where(gt, nll, 0.0), axis=(1, 2))[:, None]
        c_eq = c_eq + jnp.sum(eq.astype(jnp.float32), axis=(1, 2))[:, None]
        s_eq = s_eq + jnp.sum(jnp.where(eq, nll, 0.0), axis=(1, 2))[:, None]
        return m, s_gt, c_eq, s_eq

    z = jnp.zeros((B, 1), jnp.float32)
    m, s_gt, c_eq, s_eq = jax.lax.fori_loop(0, nchunk, stats, (z, z, z, z))
    contrib = s_gt + (_KEEP - m) * s_eq / c_eq
    out_ref[0, 0] = jnp.sum(contrib) / (B * _KEEP)


def kernel(inputs, targets):
    B, C, H, W = inputs.shape
    grid1 = (B, H // _RB)
    dist, nll = pl.pallas_call(
        _stage1_body,
        grid=grid1,
        in_specs=[
            pl.BlockSpec((1, C, _RB, W), lambda b, i: (b, 0, i, 0)),
            pl.BlockSpec((1, _RB, W), lambda b, i: (b, i, 0)),
        ],
        out_specs=[
            pl.BlockSpec((1, _RB, W), lambda b, i: (b, i, 0)),
            pl.BlockSpec((1, _RB, W), lambda b, i: (b, i, 0)),
        ],
        out_shape=[
            jax.ShapeDtypeStruct((B, H, W), jnp.float32),
            jax.ShapeDtypeStruct((B, H, W), jnp.float32),
        ],
    )(inputs, targets)

    out = pl.pallas_call(
        _stage2_body,
        out_specs=pl.BlockSpec(memory_space=pltpu.SMEM),
        out_shape=jax.ShapeDtypeStruct((1, 1), jnp.float32),
    )(dist, nll)
    return out[0, 0]
